# baseline (speedup 1.0000x reference)
import functools

import jax
import jax.numpy as jnp
from jax import lax
from jax.experimental import pallas as pl
from jax.experimental.pallas import tpu as pltpu
from jax.experimental.pallas import tpu_sc as plsc

N = 10000
N_PAD = 10240
E = 320000
D = 128
C = 64
NC = 2
NS = 16
NW = NC * NS
NB = 4
ITERS = 160
E_PER_W = ITERS * C
E_PAD = NW * E_PER_W
RPS = N_PAD // NS

_mesh = plsc.VectorSubcoreMesh(core_axis_name="c", subcore_axis_name="s")
_f32 = jnp.float32


def _wid():
    return lax.axis_index("s") * NC + lax.axis_index("c")


@functools.partial(
    pl.kernel,
    out_type=jax.ShapeDtypeStruct((NC, N_PAD, D), _f32),
    mesh=_mesh,
    scratch_types=[
        pltpu.VMEM((2, C), jnp.int32),
        pltpu.VMEM((2, C), jnp.int32),
        pltpu.VMEM((2, C), jnp.int32),
        pltpu.VMEM((2, C), jnp.int32),
        pltpu.VMEM((C, D), _f32),
        pltpu.VMEM_SHARED((N_PAD, D), _f32),
        pltpu.SemaphoreType.DMA,
        pltpu.SemaphoreType.DMA,
        pltpu.SemaphoreType.DMA,
        pltpu.SemaphoreType.DMA,
    ],
)
def _deg_kernel(edges_hbm, zeros_hbm, ones_hbm, out_hbm,
                i0, i1, i2, i3, ones_v, acc, s0, s1, s2, s3):
    IDX = [i0, i1, i2, i3]
    SI = [s0, s1, s2, s3]
    cid = lax.axis_index("c")
    sid = lax.axis_index("s")
    wid = _wid()
    rbase = sid * RPS

    pltpu.sync_copy(ones_hbm, ones_v)
    pltpu.sync_copy(zeros_hbm.at[pl.ds(rbase, RPS)], acc.at[pl.ds(rbase, RPS)])
    for b in range(NB):
        pltpu.async_copy(edges_hbm.at[wid + b * NW], IDX[b], SI[b])
    plsc.subcore_barrier()

    @pl.loop(0, ITERS, step=NB)
    def _(k):
        for b in range(NB):
            kk = k + b
            pltpu.make_async_copy(edges_hbm.at[wid], IDX[b], SI[b]).wait()
            pltpu.sync_copy(ones_v, acc.at[IDX[b].at[1]], add=True)

            @pl.when(kk + NB < ITERS)
            def _():
                pltpu.async_copy(edges_hbm.at[wid + (kk + NB) * NW], IDX[b], SI[b])

    plsc.subcore_barrier()
    pltpu.sync_copy(acc.at[pl.ds(rbase, RPS)], out_hbm.at[cid, pl.ds(rbase, RPS)])


@functools.partial(
    pl.kernel,
    out_type=jax.ShapeDtypeStruct((NC, N_PAD, D), _f32),
    mesh=_mesh,
    scratch_types=[
        pltpu.VMEM((2, C), jnp.int32),
        pltpu.VMEM((2, C), jnp.int32),
        pltpu.VMEM((2, C), jnp.int32),
        pltpu.VMEM((2, C), jnp.int32),
        pltpu.VMEM((C, D), _f32),
        pltpu.VMEM((C, D), _f32),
        pltpu.VMEM((C, D), _f32),
        pltpu.VMEM((C, D), _f32),
        pltpu.VMEM_SHARED((N_PAD, D), _f32),
        pltpu.SemaphoreType.DMA,
        pltpu.SemaphoreType.DMA,
        pltpu.SemaphoreType.DMA,
        pltpu.SemaphoreType.DMA,
        pltpu.SemaphoreType.DMA,
        pltpu.SemaphoreType.DMA,
        pltpu.SemaphoreType.DMA,
        pltpu.SemaphoreType.DMA,
    ],
)
def _hop_kernel(y_hbm, edges_hbm, zeros_hbm, out_hbm,
                i0, i1, i2, i3, r0, r1, r2, r3, acc,
                s0, s1, s2, s3, g0, g1, g2, g3):
    IDX = [i0, i1, i2, i3]
    SI = [s0, s1, s2, s3]
    ROWS = [r0, r1, r2, r3]
    SG = [g0, g1, g2, g3]
    cid = lax.axis_index("c")
    sid = lax.axis_index("s")
    wid = _wid()
    rbase = sid * RPS

    pltpu.sync_copy(zeros_hbm.at[pl.ds(rbase, RPS)], acc.at[pl.ds(rbase, RPS)])
    for b in range(NB):
        pltpu.async_copy(edges_hbm.at[wid + b * NW], IDX[b], SI[b])
    plsc.subcore_barrier()

    for b in range(NB - 1):
        pltpu.make_async_copy(edges_hbm.at[wid], IDX[b], SI[b]).wait()
        pltpu.async_copy(y_hbm.at[IDX[b].at[0]], ROWS[b], SG[b])

    @pl.loop(0, ITERS, step=NB)
    def _(k):
        for b in range(NB):
            kk = k + b
            nb = (b + NB - 1) % NB
            pltpu.make_async_copy(y_hbm.at[pl.ds(0, C)], ROWS[b], SG[b]).wait()

            @pl.when(kk + NB - 1 < ITERS)
            def _():
                pltpu.make_async_copy(edges_hbm.at[wid], IDX[nb], SI[nb]).wait()
                pltpu.async_copy(y_hbm.at[IDX[nb].at[0]], ROWS[nb], SG[nb])

            pltpu.sync_copy(ROWS[b], acc.at[IDX[b].at[1]], add=True)

            @pl.when(kk + NB < ITERS)
            def _():
                pltpu.async_copy(edges_hbm.at[wid + (kk + NB) * NW], IDX[b], SI[b])

    plsc.subcore_barrier()
    pltpu.sync_copy(acc.at[pl.ds(rbase, RPS)], out_hbm.at[cid, pl.ds(rbase, RPS)])


def _tc_scalings(d0, d1, x_pad):
    def body(d0_ref, d1_ref, x_ref, y_ref, dis_ref, inv_ref):
        deg = d0_ref[:, 0:1] + d1_ref[:, 0:1] + 1.0
        dis = lax.rsqrt(deg)
        dis_ref[...] = dis
        inv_ref[...] = 1.0 / deg
        y_ref[...] = x_ref[...] * dis

    return pl.pallas_call(
        body,
        out_shape=(
            jax.ShapeDtypeStruct((N_PAD, D), _f32),
            jax.ShapeDtypeStruct((N_PAD, 1), _f32),
            jax.ShapeDtypeStruct((N_PAD, 1), _f32),
        ),
    )(d0, d1, x_pad)


def _tc_mid(q0, q1, y1, inv):
    def body(q0_ref, q1_ref, y1_ref, inv_ref, y2_ref):
        z = q0_ref[...] + q1_ref[...] + y1_ref[...]
        y2_ref[...] = z * inv_ref[...]

    return pl.pallas_call(
        body, out_shape=jax.ShapeDtypeStruct((N_PAD, D), _f32),
    )(q0, q1, y1, inv)


def _tc_final(r0, r1, y2, dis, W, b2):
    def body(r0_ref, r1_ref, y2_ref, dis_ref, w_ref, b_ref, out_ref):
        h = (r0_ref[...] + r1_ref[...] + y2_ref[...]) * dis_ref[...]
        out_ref[...] = lax.dot_general(
            h, w_ref[...], (((1,), (1,)), ((), ())),
            preferred_element_type=_f32,
        ) + b_ref[...]

    return pl.pallas_call(
        body, out_shape=jax.ShapeDtypeStruct((N_PAD, D), _f32),
    )(r0, r1, y2, dis, W, b2)


def kernel(edge_index, x, W, b):
    src = edge_index[0].astype(jnp.int32)
    dst = edge_index[1].astype(jnp.int32)
    pad = N + (jnp.arange(E_PAD - E, dtype=jnp.int32) % (N_PAD - N))
    sp = jnp.concatenate([src, pad]).reshape(NW * ITERS, 1, C)
    dp = jnp.concatenate([dst, pad]).reshape(NW * ITERS, 1, C)
    edges = jnp.concatenate([sp, dp], axis=1)
    x_pad = jnp.pad(x, ((0, N_PAD - N), (0, 0)))
    zeros_d = jnp.zeros((N_PAD, D), _f32)

    dparts = _deg_kernel(edges, zeros_d, jnp.ones((C, D), _f32))
    y1, dis, inv = _tc_scalings(dparts[0], dparts[1], x_pad)
    qparts = _hop_kernel(y1, edges, zeros_d)
    y2 = _tc_mid(qparts[0], qparts[1], y1, inv)
    rparts = _hop_kernel(y2, edges, zeros_d)
    out = _tc_final(rparts[0], rparts[1], y2, dis, W, b.reshape(1, D))
    return out[:N]


# speedup vs baseline: 1.2041x; 1.2041x over previous
import functools

import jax
import jax.numpy as jnp
from jax import lax
from jax.experimental import pallas as pl
from jax.experimental.pallas import tpu as pltpu
from jax.experimental.pallas import tpu_sc as plsc

N = 10000
N_PAD = 10240
E = 320000
D = 128
C = 128
NC = 2
NS = 16
NW = NC * NS
NB = 4
ITERS = 80
E_PER_W = ITERS * C
E_PAD = NW * E_PER_W
RPS = N_PAD // NS

_mesh = plsc.VectorSubcoreMesh(core_axis_name="c", subcore_axis_name="s")
_f32 = jnp.float32


def _wid():
    return lax.axis_index("s") * NC + lax.axis_index("c")


def _fill(buf, val):
    v = jnp.full((16,), val, _f32)

    @pl.loop(0, C)
    def _(r):
        for j in range(D // 16):
            buf[r, pl.ds(j * 16, 16)] = v


def _zero_acc(buf, acc, rbase):
    _fill(buf, 0.0)

    @pl.loop(0, RPS // C)
    def _(p):
        pltpu.sync_copy(buf, acc.at[pl.ds(rbase + p * C, C)])


@functools.partial(
    pl.kernel,
    out_type=jax.ShapeDtypeStruct((NC, N_PAD, D), _f32),
    mesh=_mesh,
    scratch_types=[
        pltpu.VMEM((2, C), jnp.int32),
        pltpu.VMEM((2, C), jnp.int32),
        pltpu.VMEM((2, C), jnp.int32),
        pltpu.VMEM((2, C), jnp.int32),
        pltpu.VMEM((C, D), _f32),
        pltpu.VMEM_SHARED((N_PAD, D), _f32),
        pltpu.SemaphoreType.DMA,
        pltpu.SemaphoreType.DMA,
        pltpu.SemaphoreType.DMA,
        pltpu.SemaphoreType.DMA,
    ],
)
def _deg_kernel(edges_hbm, out_hbm,
                i0, i1, i2, i3, ones_v, acc, s0, s1, s2, s3):
    IDX = [i0, i1, i2, i3]
    SI = [s0, s1, s2, s3]
    cid = lax.axis_index("c")
    sid = lax.axis_index("s")
    wid = _wid()
    rbase = sid * RPS

    _zero_acc(ones_v, acc, rbase)
    _fill(ones_v, 1.0)
    for b in range(NB):
        pltpu.async_copy(edges_hbm.at[wid + b * NW], IDX[b], SI[b])
    plsc.subcore_barrier()

    @pl.loop(0, ITERS, step=NB)
    def _(k):
        for b in range(NB):
            kk = k + b
            pltpu.make_async_copy(edges_hbm.at[wid], IDX[b], SI[b]).wait()
            pltpu.sync_copy(ones_v, acc.at[IDX[b].at[1]], add=True)

            @pl.when(kk + NB < ITERS)
            def _():
                pltpu.async_copy(edges_hbm.at[wid + (kk + NB) * NW], IDX[b], SI[b])

    plsc.subcore_barrier()
    pltpu.sync_copy(acc.at[pl.ds(rbase, RPS)], out_hbm.at[cid, pl.ds(rbase, RPS)])


@functools.partial(
    pl.kernel,
    out_type=jax.ShapeDtypeStruct((NC, N_PAD, D), _f32),
    mesh=_mesh,
    scratch_types=[
        pltpu.VMEM((2, C), jnp.int32),
        pltpu.VMEM((2, C), jnp.int32),
        pltpu.VMEM((2, C), jnp.int32),
        pltpu.VMEM((2, C), jnp.int32),
        pltpu.VMEM((C, D), _f32),
        pltpu.VMEM((C, D), _f32),
        pltpu.VMEM_SHARED((N_PAD, D), _f32),
        pltpu.SemaphoreType.DMA,
        pltpu.SemaphoreType.DMA,
        pltpu.SemaphoreType.DMA,
        pltpu.SemaphoreType.DMA,
        pltpu.SemaphoreType.DMA,
        pltpu.SemaphoreType.DMA,
    ],
)
def _hop_kernel(y_hbm, edges_hbm, out_hbm,
                i0, i1, i2, i3, r0, r1, acc, s0, s1, s2, s3, g0, g1):
    IDX = [i0, i1, i2, i3]
    SI = [s0, s1, s2, s3]
    ROWS = [r0, r1]
    SG = [g0, g1]
    cid = lax.axis_index("c")
    sid = lax.axis_index("s")
    wid = _wid()
    rbase = sid * RPS

    for b in range(NB):
        pltpu.async_copy(edges_hbm.at[wid + b * NW], IDX[b], SI[b])
    _zero_acc(ROWS[0], acc, rbase)
    plsc.subcore_barrier()

    pltpu.make_async_copy(edges_hbm.at[wid], IDX[0], SI[0]).wait()
    pltpu.async_copy(y_hbm.at[IDX[0].at[0]], ROWS[0], SG[0])

    @pl.loop(0, ITERS, step=NB)
    def _(k):
        for b in range(NB):
            kk = k + b
            rb = b % 2
            nb = (b + 1) % NB
            nrb = (b + 1) % 2
            pltpu.make_async_copy(y_hbm.at[pl.ds(0, C)], ROWS[rb], SG[rb]).wait()

            @pl.when(kk + 1 < ITERS)
            def _():
                pltpu.make_async_copy(edges_hbm.at[wid], IDX[nb], SI[nb]).wait()
                pltpu.async_copy(y_hbm.at[IDX[nb].at[0]], ROWS[nrb], SG[nrb])

            pltpu.sync_copy(ROWS[rb], acc.at[IDX[b].at[1]], add=True)

            @pl.when(kk + NB < ITERS)
            def _():
                pltpu.async_copy(edges_hbm.at[wid + (kk + NB) * NW], IDX[b], SI[b])

    plsc.subcore_barrier()
    pltpu.sync_copy(acc.at[pl.ds(rbase, RPS)], out_hbm.at[cid, pl.ds(rbase, RPS)])


def _tc_scalings(d0, d1, x_pad):
    def body(d0_ref, d1_ref, x_ref, y_ref, dis_ref, inv_ref):
        deg = d0_ref[:, 0:1] + d1_ref[:, 0:1] + 1.0
        dis = lax.rsqrt(deg)
        dis_ref[...] = dis
        inv_ref[...] = 1.0 / deg
        y_ref[...] = x_ref[...] * dis

    return pl.pallas_call(
        body,
        out_shape=(
            jax.ShapeDtypeStruct((N_PAD, D), _f32),
            jax.ShapeDtypeStruct((N_PAD, 1), _f32),
            jax.ShapeDtypeStruct((N_PAD, 1), _f32),
        ),
    )(d0, d1, x_pad)


def _tc_mid(q0, q1, y1, inv):
    def body(q0_ref, q1_ref, y1_ref, inv_ref, y2_ref):
        z = q0_ref[...] + q1_ref[...] + y1_ref[...]
        y2_ref[...] = z * inv_ref[...]

    return pl.pallas_call(
        body, out_shape=jax.ShapeDtypeStruct((N_PAD, D), _f32),
    )(q0, q1, y1, inv)


def _tc_final(r0, r1, y2, dis, W, b2):
    def body(r0_ref, r1_ref, y2_ref, dis_ref, w_ref, b_ref, out_ref):
        h = (r0_ref[...] + r1_ref[...] + y2_ref[...]) * dis_ref[...]
        out_ref[...] = lax.dot_general(
            h, w_ref[...], (((1,), (1,)), ((), ())),
            preferred_element_type=_f32,
        ) + b_ref[...]

    return pl.pallas_call(
        body, out_shape=jax.ShapeDtypeStruct((N_PAD, D), _f32),
    )(r0, r1, y2, dis, W, b2)


def kernel(edge_index, x, W, b):
    src = edge_index[0].astype(jnp.int32)
    dst = edge_index[1].astype(jnp.int32)
    pad = N + (jnp.arange(E_PAD - E, dtype=jnp.int32) % (N_PAD - N))
    sp = jnp.concatenate([src, pad]).reshape(NW * ITERS, 1, C)
    dp = jnp.concatenate([dst, pad]).reshape(NW * ITERS, 1, C)
    edges = jnp.concatenate([sp, dp], axis=1)
    x_pad = jnp.pad(x, ((0, N_PAD - N), (0, 0)))

    dparts = _deg_kernel(edges)
    y1, dis, inv = _tc_scalings(dparts[0], dparts[1], x_pad)
    qparts = _hop_kernel(y1, edges)
    y2 = _tc_mid(qparts[0], qparts[1], y1, inv)
    rparts = _hop_kernel(y2, edges)
    out = _tc_final(rparts[0], rparts[1], y2, dis, W, b.reshape(1, D))
    return out[:N]


# speedup vs baseline: 1.2843x; 1.0667x over previous
import functools

import jax
import jax.numpy as jnp
from jax import lax
from jax.experimental import pallas as pl
from jax.experimental.pallas import tpu as pltpu
from jax.experimental.pallas import tpu_sc as plsc

N = 10000
N_PAD = 10240
E = 320000
D = 128
C = 128
NC = 2
NS = 16
NW = NC * NS
NB = 4
ITERS = 80
E_PER_W = ITERS * C
E_PAD = NW * E_PER_W
RPS = N_PAD // NS

_mesh = plsc.VectorSubcoreMesh(core_axis_name="c", subcore_axis_name="s")
_f32 = jnp.float32


def _wid():
    return lax.axis_index("s") * NC + lax.axis_index("c")


DEG_W = 64


def _fill(buf, val):
    v = jnp.full((16,), val, _f32)

    @pl.loop(0, C)
    def _(r):
        for j in range(buf.shape[1] // 16):
            buf[r, pl.ds(j * 16, 16)] = v


def _zero_acc(buf, acc, rbase):
    _fill(buf, 0.0)

    @pl.loop(0, RPS // C)
    def _(p):
        pltpu.sync_copy(buf, acc.at[pl.ds(rbase + p * C, C)])


@functools.partial(
    pl.kernel,
    out_type=jax.ShapeDtypeStruct((NC, N_PAD, DEG_W), _f32),
    mesh=_mesh,
    scratch_types=[
        pltpu.VMEM((2, C), jnp.int32),
        pltpu.VMEM((2, C), jnp.int32),
        pltpu.VMEM((2, C), jnp.int32),
        pltpu.VMEM((2, C), jnp.int32),
        pltpu.VMEM((C, DEG_W), _f32),
        pltpu.VMEM_SHARED((N_PAD, DEG_W), _f32),
        pltpu.SemaphoreType.DMA,
        pltpu.SemaphoreType.DMA,
        pltpu.SemaphoreType.DMA,
        pltpu.SemaphoreType.DMA,
    ],
)
def _deg_kernel(edges_hbm, out_hbm,
                i0, i1, i2, i3, ones_v, acc, s0, s1, s2, s3):
    IDX = [i0, i1, i2, i3]
    SI = [s0, s1, s2, s3]
    cid = lax.axis_index("c")
    sid = lax.axis_index("s")
    wid = _wid()
    rbase = sid * RPS

    _zero_acc(ones_v, acc, rbase)
    _fill(ones_v, 1.0)
    for b in range(NB):
        pltpu.async_copy(edges_hbm.at[wid + b * NW], IDX[b], SI[b])
    plsc.subcore_barrier()

    @pl.loop(0, ITERS, step=NB)
    def _(k):
        for b in range(NB):
            kk = k + b
            pltpu.make_async_copy(edges_hbm.at[wid], IDX[b], SI[b]).wait()
            pltpu.sync_copy(ones_v, acc.at[IDX[b].at[1]], add=True)

            @pl.when(kk + NB < ITERS)
            def _():
                pltpu.async_copy(edges_hbm.at[wid + (kk + NB) * NW], IDX[b], SI[b])

    plsc.subcore_barrier()
    pltpu.sync_copy(acc.at[pl.ds(rbase, RPS)], out_hbm.at[cid, pl.ds(rbase, RPS)])


@functools.partial(
    pl.kernel,
    out_type=jax.ShapeDtypeStruct((NC, N_PAD, D), _f32),
    mesh=_mesh,
    scratch_types=[
        pltpu.VMEM((2, C), jnp.int32),
        pltpu.VMEM((2, C), jnp.int32),
        pltpu.VMEM((2, C), jnp.int32),
        pltpu.VMEM((2, C), jnp.int32),
        pltpu.VMEM((C, D), _f32),
        pltpu.VMEM((C, D), _f32),
        pltpu.VMEM_SHARED((N_PAD, D), _f32),
        pltpu.SemaphoreType.DMA,
        pltpu.SemaphoreType.DMA,
        pltpu.SemaphoreType.DMA,
        pltpu.SemaphoreType.DMA,
        pltpu.SemaphoreType.DMA,
        pltpu.SemaphoreType.DMA,
    ],
)
def _hop_kernel(y_hbm, edges_hbm, out_hbm,
                i0, i1, i2, i3, r0, r1, acc, s0, s1, s2, s3, g0, g1):
    IDX = [i0, i1, i2, i3]
    SI = [s0, s1, s2, s3]
    ROWS = [r0, r1]
    SG = [g0, g1]
    cid = lax.axis_index("c")
    sid = lax.axis_index("s")
    wid = _wid()
    rbase = sid * RPS

    for b in range(NB):
        pltpu.async_copy(edges_hbm.at[wid + b * NW], IDX[b], SI[b])
    _zero_acc(ROWS[0], acc, rbase)
    plsc.subcore_barrier()

    pltpu.make_async_copy(edges_hbm.at[wid], IDX[0], SI[0]).wait()
    pltpu.async_copy(y_hbm.at[IDX[0].at[0]], ROWS[0], SG[0])

    @pl.loop(0, ITERS, step=NB)
    def _(k):
        for b in range(NB):
            kk = k + b
            rb = b % 2
            nb = (b + 1) % NB
            nrb = (b + 1) % 2
            pltpu.make_async_copy(y_hbm.at[pl.ds(0, C)], ROWS[rb], SG[rb]).wait()

            @pl.when(kk + 1 < ITERS)
            def _():
                pltpu.make_async_copy(edges_hbm.at[wid], IDX[nb], SI[nb]).wait()
                pltpu.async_copy(y_hbm.at[IDX[nb].at[0]], ROWS[nrb], SG[nrb])

            pltpu.sync_copy(ROWS[rb], acc.at[IDX[b].at[1]], add=True)

            @pl.when(kk + NB < ITERS)
            def _():
                pltpu.async_copy(edges_hbm.at[wid + (kk + NB) * NW], IDX[b], SI[b])

    plsc.subcore_barrier()
    pltpu.sync_copy(acc.at[pl.ds(rbase, RPS)], out_hbm.at[cid, pl.ds(rbase, RPS)])


def _tc_scalings(d0, d1, x_pad):
    def body(d0_ref, d1_ref, x_ref, y_ref, dis_ref, inv_ref):
        deg = d0_ref[:, 0:1] + d1_ref[:, 0:1] + 1.0
        dis = lax.rsqrt(deg)
        dis_ref[...] = dis
        inv_ref[...] = 1.0 / deg
        y_ref[...] = x_ref[...] * dis

    return pl.pallas_call(
        body,
        out_shape=(
            jax.ShapeDtypeStruct((N_PAD, D), _f32),
            jax.ShapeDtypeStruct((N_PAD, 1), _f32),
            jax.ShapeDtypeStruct((N_PAD, 1), _f32),
        ),
    )(d0, d1, x_pad)


def _tc_mid(q0, q1, y1, inv):
    def body(q0_ref, q1_ref, y1_ref, inv_ref, y2_ref):
        z = q0_ref[...] + q1_ref[...] + y1_ref[...]
        y2_ref[...] = z * inv_ref[...]

    return pl.pallas_call(
        body, out_shape=jax.ShapeDtypeStruct((N_PAD, D), _f32),
    )(q0, q1, y1, inv)


def _tc_final(r0, r1, y2, dis, W, b2):
    def body(r0_ref, r1_ref, y2_ref, dis_ref, w_ref, b_ref, out_ref):
        h = (r0_ref[...] + r1_ref[...] + y2_ref[...]) * dis_ref[...]
        out_ref[...] = lax.dot_general(
            h, w_ref[...], (((1,), (1,)), ((), ())),
            preferred_element_type=_f32,
        ) + b_ref[...]

    return pl.pallas_call(
        body, out_shape=jax.ShapeDtypeStruct((N_PAD, D), _f32),
    )(r0, r1, y2, dis, W, b2)


def kernel(edge_index, x, W, b):
    src = edge_index[0].astype(jnp.int32)
    dst = edge_index[1].astype(jnp.int32)
    pad = N + (jnp.arange(E_PAD - E, dtype=jnp.int32) % (N_PAD - N))
    sp = jnp.concatenate([src, pad]).reshape(NW * ITERS, 1, C)
    dp = jnp.concatenate([dst, pad]).reshape(NW * ITERS, 1, C)
    edges = jnp.concatenate([sp, dp], axis=1)
    x_pad = jnp.pad(x, ((0, N_PAD - N), (0, 0)))

    dparts = _deg_kernel(edges)
    y1, dis, inv = _tc_scalings(dparts[0], dparts[1], x_pad)
    qparts = _hop_kernel(y1, edges)
    y2 = _tc_mid(qparts[0], qparts[1], y1, inv)
    rparts = _hop_kernel(y2, edges)
    out = _tc_final(rparts[0], rparts[1], y2, dis, W, b.reshape(1, D))
    return out[:N]


# speedup vs baseline: 1.2995x; 1.0118x over previous
import functools

import jax
import jax.numpy as jnp
from jax import lax
from jax.experimental import pallas as pl
from jax.experimental.pallas import tpu as pltpu
from jax.experimental.pallas import tpu_sc as plsc

N = 10000
N_PAD = 10240
E = 320000
D = 128
C = 128
NC = 2
NS = 16
NW = NC * NS
NB = 4
ITERS = 80
E_PER_W = ITERS * C
E_PAD = NW * E_PER_W
RPS = N_PAD // NS

_mesh = plsc.VectorSubcoreMesh(core_axis_name="c", subcore_axis_name="s")
_f32 = jnp.float32


def _wid():
    return lax.axis_index("s") * NC + lax.axis_index("c")


DEG_W = 64


def _fill(buf, val):
    v = jnp.full((16,), val, _f32)

    @pl.loop(0, C)
    def _(r):
        for j in range(buf.shape[1] // 16):
            buf[r, pl.ds(j * 16, 16)] = v


def _zero_acc(buf, acc, rbase):
    _fill(buf, 0.0)

    @pl.loop(0, RPS // C)
    def _(p):
        pltpu.sync_copy(buf, acc.at[pl.ds(rbase + p * C, C)])


@functools.partial(
    pl.kernel,
    out_type=jax.ShapeDtypeStruct((NC, N_PAD, DEG_W), _f32),
    mesh=_mesh,
    scratch_types=[
        pltpu.VMEM((2, C), jnp.int32),
        pltpu.VMEM((2, C), jnp.int32),
        pltpu.VMEM((2, C), jnp.int32),
        pltpu.VMEM((2, C), jnp.int32),
        pltpu.VMEM((C, DEG_W), _f32),
        pltpu.VMEM_SHARED((N_PAD, DEG_W), _f32),
        pltpu.SemaphoreType.DMA,
        pltpu.SemaphoreType.DMA,
        pltpu.SemaphoreType.DMA,
        pltpu.SemaphoreType.DMA,
    ],
)
def _deg_kernel(edges_hbm, out_hbm,
                i0, i1, i2, i3, ones_v, acc, s0, s1, s2, s3):
    IDX = [i0, i1, i2, i3]
    SI = [s0, s1, s2, s3]
    cid = lax.axis_index("c")
    sid = lax.axis_index("s")
    wid = _wid()
    rbase = sid * RPS

    _zero_acc(ones_v, acc, rbase)
    _fill(ones_v, 1.0)
    for b in range(NB):
        pltpu.async_copy(edges_hbm.at[wid + b * NW], IDX[b], SI[b])
    plsc.subcore_barrier()

    @pl.loop(0, ITERS, step=NB)
    def _(k):
        for b in range(NB):
            kk = k + b
            pltpu.make_async_copy(edges_hbm.at[wid], IDX[b], SI[b]).wait()
            pltpu.sync_copy(ones_v, acc.at[IDX[b].at[1]], add=True)

            @pl.when(kk + NB < ITERS)
            def _():
                pltpu.async_copy(edges_hbm.at[wid + (kk + NB) * NW], IDX[b], SI[b])

    plsc.subcore_barrier()
    pltpu.sync_copy(acc.at[pl.ds(rbase, RPS)], out_hbm.at[cid, pl.ds(rbase, RPS)])


@functools.partial(
    pl.kernel,
    out_type=jax.ShapeDtypeStruct((NC, N_PAD, D), _f32),
    mesh=_mesh,
    scratch_types=[
        pltpu.VMEM((2, C), jnp.int32),
        pltpu.VMEM((2, C), jnp.int32),
        pltpu.VMEM((2, C), jnp.int32),
        pltpu.VMEM((2, C), jnp.int32),
        pltpu.VMEM((C, D), _f32),
        pltpu.VMEM((C, D), _f32),
        pltpu.VMEM_SHARED((N_PAD, D), _f32),
        pltpu.SemaphoreType.DMA,
        pltpu.SemaphoreType.DMA,
        pltpu.SemaphoreType.DMA,
        pltpu.SemaphoreType.DMA,
        pltpu.SemaphoreType.DMA,
        pltpu.SemaphoreType.DMA,
        pltpu.SemaphoreType.DMA,
        pltpu.SemaphoreType.DMA,
    ],
)
def _hop_kernel(y_hbm, edges_hbm, out_hbm,
                i0, i1, i2, i3, r0, r1, acc,
                s0, s1, s2, s3, g0, g1, t0, t1):
    IDX = [i0, i1, i2, i3]
    SI = [s0, s1, s2, s3]
    ROWS = [r0, r1]
    SG = [g0, g1]
    SS = [t0, t1]
    cid = lax.axis_index("c")
    sid = lax.axis_index("s")
    wid = _wid()
    rbase = sid * RPS

    for b in range(NB):
        pltpu.async_copy(edges_hbm.at[wid + b * NW], IDX[b], SI[b])
    _zero_acc(ROWS[0], acc, rbase)
    plsc.subcore_barrier()

    pltpu.make_async_copy(edges_hbm.at[wid], IDX[0], SI[0]).wait()
    pltpu.async_copy(y_hbm.at[IDX[0].at[0]], ROWS[0], SG[0])

    @pl.loop(0, ITERS, step=NB)
    def _(k):
        for b in range(NB):
            kk = k + b
            a = b % 2
            o = (b + 1) % 2
            nb = (b + 1) % NB
            pltpu.make_async_copy(y_hbm.at[pl.ds(0, C)], ROWS[a], SG[a]).wait()

            pltpu.async_copy(ROWS[a], acc.at[IDX[b].at[1]], SS[a], add=True)

            @pl.when(kk > 0)
            def _():
                pltpu.make_async_copy(
                    ROWS[o], acc.at[IDX[(b + NB - 1) % NB].at[1]], SS[o]).wait()

            @pl.when(kk + 1 < ITERS)
            def _():
                pltpu.make_async_copy(edges_hbm.at[wid], IDX[nb], SI[nb]).wait()
                pltpu.async_copy(y_hbm.at[IDX[nb].at[0]], ROWS[o], SG[o])

            @pl.when((kk > 0) & (kk + NB - 1 < ITERS))
            def _():
                pltpu.async_copy(
                    edges_hbm.at[wid + (kk + NB - 1) * NW], IDX[(b + NB - 1) % NB],
                    SI[(b + NB - 1) % NB])

    pltpu.make_async_copy(ROWS[(ITERS - 1) % 2],
                          acc.at[IDX[(ITERS - 1) % NB].at[1]],
                          SS[(ITERS - 1) % 2]).wait()
    plsc.subcore_barrier()
    pltpu.sync_copy(acc.at[pl.ds(rbase, RPS)], out_hbm.at[cid, pl.ds(rbase, RPS)])


def _tc_scalings(d0, d1, x_pad):
    def body(d0_ref, d1_ref, x_ref, y_ref, dis_ref, inv_ref):
        deg = d0_ref[:, 0:1] + d1_ref[:, 0:1] + 1.0
        dis = lax.rsqrt(deg)
        dis_ref[...] = dis
        inv_ref[...] = 1.0 / deg
        y_ref[...] = x_ref[...] * dis

    return pl.pallas_call(
        body,
        out_shape=(
            jax.ShapeDtypeStruct((N_PAD, D), _f32),
            jax.ShapeDtypeStruct((N_PAD, 1), _f32),
            jax.ShapeDtypeStruct((N_PAD, 1), _f32),
        ),
    )(d0, d1, x_pad)


def _tc_mid(q0, q1, y1, inv):
    def body(q0_ref, q1_ref, y1_ref, inv_ref, y2_ref):
        z = q0_ref[...] + q1_ref[...] + y1_ref[...]
        y2_ref[...] = z * inv_ref[...]

    return pl.pallas_call(
        body, out_shape=jax.ShapeDtypeStruct((N_PAD, D), _f32),
    )(q0, q1, y1, inv)


def _tc_final(r0, r1, y2, dis, W, b2):
    def body(r0_ref, r1_ref, y2_ref, dis_ref, w_ref, b_ref, out_ref):
        h = ((r0_ref[:N, :] + r1_ref[:N, :] + y2_ref[:N, :])
             * dis_ref[:N, :])
        out_ref[...] = lax.dot_general(
            h, w_ref[...], (((1,), (1,)), ((), ())),
            preferred_element_type=_f32,
        ) + b_ref[...]

    return pl.pallas_call(
        body, out_shape=jax.ShapeDtypeStruct((N, D), _f32),
    )(r0, r1, y2, dis, W, b2)


def kernel(edge_index, x, W, b):
    src = edge_index[0].astype(jnp.int32)
    dst = edge_index[1].astype(jnp.int32)
    pad = N + (jnp.arange(E_PAD - E, dtype=jnp.int32) % (N_PAD - N))
    sp = jnp.concatenate([src, pad]).reshape(NW * ITERS, 1, C)
    dp = jnp.concatenate([dst, pad]).reshape(NW * ITERS, 1, C)
    edges = jnp.concatenate([sp, dp], axis=1)
    x_pad = jnp.pad(x, ((0, N_PAD - N), (0, 0)))

    dparts = _deg_kernel(edges)
    y1, dis, inv = _tc_scalings(dparts[0], dparts[1], x_pad)
    qparts = _hop_kernel(y1, edges)
    y2 = _tc_mid(qparts[0], qparts[1], y1, inv)
    rparts = _hop_kernel(y2, edges)
    return _tc_final(rparts[0], rparts[1], y2, dis, W, b.reshape(1, D))


# speedup vs baseline: 1.3002x; 1.0006x over previous
import functools

import jax
import jax.numpy as jnp
from jax import lax
from jax.experimental import pallas as pl
from jax.experimental.pallas import tpu as pltpu
from jax.experimental.pallas import tpu_sc as plsc

N = 10000
N_PAD = 10240
E = 320000
D = 128
C = 128
NC = 2
NS = 16
NW = NC * NS
NB = 4
ITERS = 80
E_PER_W = ITERS * C
E_PAD = NW * E_PER_W
RPS = N_PAD // NS

_mesh = plsc.VectorSubcoreMesh(core_axis_name="c", subcore_axis_name="s")
_f32 = jnp.float32


def _wid():
    return lax.axis_index("s") * NC + lax.axis_index("c")


DEG_W = 64


def _fill(buf, val):
    v = jnp.full((16,), val, _f32)

    @pl.loop(0, C)
    def _(r):
        for j in range(buf.shape[1] // 16):
            buf[r, pl.ds(j * 16, 16)] = v


def _zero_acc(buf, acc, rbase):
    _fill(buf, 0.0)

    @pl.loop(0, RPS // C)
    def _(p):
        pltpu.sync_copy(buf, acc.at[pl.ds(rbase + p * C, C)])


@functools.partial(
    pl.kernel,
    out_type=jax.ShapeDtypeStruct((NC, N_PAD, DEG_W), _f32),
    mesh=_mesh,
    scratch_types=[
        pltpu.VMEM((2, C), jnp.int32),
        pltpu.VMEM((2, C), jnp.int32),
        pltpu.VMEM((2, C), jnp.int32),
        pltpu.VMEM((2, C), jnp.int32),
        pltpu.VMEM((C, DEG_W), _f32),
        pltpu.VMEM_SHARED((N_PAD, DEG_W), _f32),
        pltpu.SemaphoreType.DMA,
        pltpu.SemaphoreType.DMA,
        pltpu.SemaphoreType.DMA,
        pltpu.SemaphoreType.DMA,
    ],
)
def _deg_kernel(edges3_hbm, out_hbm,
                i0, i1, i2, i3, ones_v, acc, s0, s1, s2, s3):
    IDX = [i0, i1, i2, i3]
    SI = [s0, s1, s2, s3]
    cid = lax.axis_index("c")
    sid = lax.axis_index("s")
    wid = _wid()
    rbase = sid * RPS

    _zero_acc(ones_v, acc, rbase)
    _fill(ones_v, 1.0)
    for b in range(NB):
        pltpu.async_copy(edges3_hbm.at[wid + b * NW], IDX[b], SI[b])
    plsc.subcore_barrier()

    @pl.loop(0, ITERS, step=NB)
    def _(k):
        for b in range(NB):
            kk = k + b
            pltpu.make_async_copy(edges3_hbm.at[wid], IDX[b], SI[b]).wait()
            pltpu.sync_copy(ones_v, acc.at[IDX[b].at[1]], add=True)

            @pl.when(kk + NB < ITERS)
            def _():
                pltpu.async_copy(edges3_hbm.at[wid + (kk + NB) * NW], IDX[b], SI[b])

    plsc.subcore_barrier()
    pltpu.sync_copy(acc.at[pl.ds(rbase, RPS)], out_hbm.at[cid, pl.ds(rbase, RPS)])


G = 8
NG = ITERS // G


@functools.partial(
    pl.kernel,
    out_type=jax.ShapeDtypeStruct((NC, N_PAD, D), _f32),
    mesh=_mesh,
    scratch_types=[
        pltpu.VMEM((2 * G, C), jnp.int32),
        pltpu.VMEM((2 * G, C), jnp.int32),
        pltpu.VMEM((C, D), _f32),
        pltpu.VMEM((C, D), _f32),
        pltpu.VMEM_SHARED((N_PAD, D), _f32),
        pltpu.SemaphoreType.DMA,
        pltpu.SemaphoreType.DMA,
        pltpu.SemaphoreType.DMA,
        pltpu.SemaphoreType.DMA,
        pltpu.SemaphoreType.DMA,
        pltpu.SemaphoreType.DMA,
    ],
)
def _hop_kernel(y_hbm, edges_hbm, out_hbm,
                ib0, ib1, r0, r1, acc, s0, s1, g0, g1, t0, t1):
    IB = [ib0, ib1]
    SI = [s0, s1]
    ROWS = [r0, r1]
    SG = [g0, g1]
    SS = [t0, t1]
    cid = lax.axis_index("c")
    sid = lax.axis_index("s")
    wid = _wid()
    rbase = sid * RPS

    pltpu.async_copy(edges_hbm.at[wid, pl.ds(0, 2 * G)], IB[0], SI[0])
    _zero_acc(ROWS[0], acc, rbase)
    plsc.subcore_barrier()

    pltpu.make_async_copy(edges_hbm.at[wid, pl.ds(0, 2 * G)], IB[0], SI[0]).wait()
    pltpu.async_copy(y_hbm.at[IB[0].at[0]], ROWS[0], SG[0])

    @pl.loop(0, NG, step=2)
    def _(g):
        for sg in range(2):
            gg = g + sg
            os = 1 - sg
            for j in range(G):
                q = gg * G + j
                a = (G * sg + j) % 2
                o = 1 - a
                pltpu.make_async_copy(
                    y_hbm.at[pl.ds(0, C)], ROWS[a], SG[a]).wait()
                pltpu.async_copy(ROWS[a], acc.at[IB[sg].at[2 * j + 1]], SS[a],
                                 add=True)
                if j > 0:
                    pltpu.make_async_copy(
                        ROWS[o], acc.at[IB[sg].at[2 * j - 1]], SS[o]).wait()
                else:
                    @pl.when(q > 0)
                    def _():
                        pltpu.make_async_copy(
                            ROWS[o], acc.at[IB[os].at[2 * G - 1]], SS[o]).wait()

                    @pl.when(gg + 1 < NG)
                    def _():
                        pltpu.async_copy(
                            edges_hbm.at[wid, pl.ds((gg + 1) * 2 * G, 2 * G)],
                            IB[os], SI[os])

                if j < G - 1:
                    pltpu.async_copy(y_hbm.at[IB[sg].at[2 * (j + 1)]],
                                     ROWS[o], SG[o])
                else:
                    @pl.when(q + 1 < ITERS)
                    def _():
                        pltpu.make_async_copy(
                            edges_hbm.at[wid, pl.ds(0, 2 * G)], IB[os],
                            SI[os]).wait()
                        pltpu.async_copy(y_hbm.at[IB[os].at[0]],
                                         ROWS[o], SG[o])

    pltpu.make_async_copy(ROWS[(ITERS - 1) % 2],
                          acc.at[IB[(NG - 1) % 2].at[2 * G - 1]],
                          SS[(ITERS - 1) % 2]).wait()
    plsc.subcore_barrier()
    pltpu.sync_copy(acc.at[pl.ds(rbase, RPS)], out_hbm.at[cid, pl.ds(rbase, RPS)])


def _tc_scalings(d0, d1, x_pad):
    def body(d0_ref, d1_ref, x_ref, y_ref, dis_ref, inv_ref):
        deg = d0_ref[:, 0:1] + d1_ref[:, 0:1] + 1.0
        dis = lax.rsqrt(deg)
        dis_ref[...] = dis
        inv_ref[...] = 1.0 / deg
        y_ref[...] = x_ref[...] * dis

    return pl.pallas_call(
        body,
        out_shape=(
            jax.ShapeDtypeStruct((N_PAD, D), _f32),
            jax.ShapeDtypeStruct((N_PAD, 1), _f32),
            jax.ShapeDtypeStruct((N_PAD, 1), _f32),
        ),
    )(d0, d1, x_pad)


def _tc_mid(q0, q1, y1, inv):
    def body(q0_ref, q1_ref, y1_ref, inv_ref, y2_ref):
        z = q0_ref[...] + q1_ref[...] + y1_ref[...]
        y2_ref[...] = z * inv_ref[...]

    return pl.pallas_call(
        body, out_shape=jax.ShapeDtypeStruct((N_PAD, D), _f32),
    )(q0, q1, y1, inv)


def _tc_final(r0, r1, y2, dis, W, b2):
    def body(r0_ref, r1_ref, y2_ref, dis_ref, w_ref, b_ref, out_ref):
        h = ((r0_ref[:N, :] + r1_ref[:N, :] + y2_ref[:N, :])
             * dis_ref[:N, :])
        out_ref[...] = lax.dot_general(
            h, w_ref[...], (((1,), (1,)), ((), ())),
            preferred_element_type=_f32,
        ) + b_ref[...]

    return pl.pallas_call(
        body, out_shape=jax.ShapeDtypeStruct((N, D), _f32),
    )(r0, r1, y2, dis, W, b2)


def kernel(edge_index, x, W, b):
    src = edge_index[0].astype(jnp.int32)
    dst = edge_index[1].astype(jnp.int32)
    pad = N + (jnp.arange(E_PAD - E, dtype=jnp.int32) % (N_PAD - N))
    sp = jnp.concatenate([src, pad]).reshape(NW * ITERS, 1, C)
    dp = jnp.concatenate([dst, pad]).reshape(NW * ITERS, 1, C)
    edges3 = jnp.concatenate([sp, dp], axis=1)
    sp_f = jnp.concatenate([src, pad]).reshape(ITERS, NW, 1, C).transpose(1, 0, 2, 3)
    dp_f = jnp.concatenate([dst, pad]).reshape(ITERS, NW, 1, C).transpose(1, 0, 2, 3)
    edges = jnp.concatenate([sp_f, dp_f], axis=2).reshape(NW, ITERS * 2, C)
    x_pad = jnp.pad(x, ((0, N_PAD - N), (0, 0)))

    dparts = _deg_kernel(edges3)
    y1, dis, inv = _tc_scalings(dparts[0], dparts[1], x_pad)
    qparts = _hop_kernel(y1, edges)
    y2 = _tc_mid(qparts[0], qparts[1], y1, inv)
    rparts = _hop_kernel(y2, edges)
    return _tc_final(rparts[0], rparts[1], y2, dis, W, b.reshape(1, D))
